# baseline (device time: 38566 ns/iter reference)
import jax
import jax.numpy as jnp
from jax import lax
from jax.experimental import pallas as pl
from jax.experimental.pallas import tpu as pltpu

N_DEV = 4
B = 2
SQ = 128
SKV_SH = 128
HQ = 16
H_SH = 4
DH = 64
D_MODEL = 512
ROWS = B * SQ
HCOLS = H_SH * DH
SRC_DEVS = (0, 2)
NEG = -1e9


def kernel(x, Wq, K_ext, V_ext, Wo):
    x2 = x.reshape(ROWS, D_MODEL)
    k2 = K_ext.reshape(B * SKV_SH, HQ * DH)
    v2 = V_ext.reshape(B * SKV_SH, HQ * DH)

    def body(x_ref, wq_ref, k_ref, v_ref, wo_ref, out_ref,
             kg_ref, vg_ref, pg_ref,
             kv_send_sems, kv_recv_sems, p_send_sems, p_recv_sems):
        my = lax.axis_index("i")

        barrier = pltpu.get_barrier_semaphore()
        for off in range(1, N_DEV):
            pl.semaphore_signal(barrier, inc=1,
                                device_id=(lax.rem(my + off, N_DEV),),
                                device_id_type=pl.DeviceIdType.MESH)
        pl.semaphore_wait(barrier, N_DEV - 1)

        for s_idx, src in enumerate(SRC_DEVS):
            @pl.when(my == src)
            def _(s_idx=s_idx, src=src):
                kg_ref[s_idx, :, :] = k_ref[:, src * HCOLS:(src + 1) * HCOLS]
                vg_ref[s_idx, :, :] = v_ref[:, src * HCOLS:(src + 1) * HCOLS]
                d_i = 0
                for dst in range(N_DEV):
                    if dst == src:
                        continue
                    for kv, (ref, gref) in enumerate(
                            ((k_ref, kg_ref), (v_ref, vg_ref))):
                        pltpu.make_async_remote_copy(
                            src_ref=ref.at[:, pl.ds(dst * HCOLS, HCOLS)],
                            dst_ref=gref.at[s_idx],
                            send_sem=kv_send_sems.at[kv, d_i],
                            recv_sem=kv_recv_sems.at[kv, s_idx],
                            device_id=(dst,),
                            device_id_type=pl.DeviceIdType.MESH,
                        ).start()
                    d_i += 1

        q2 = jnp.dot(x_ref[:], wq_ref[:], preferred_element_type=jnp.float32)

        for s_idx, src in enumerate(SRC_DEVS):
            @pl.when(my != src)
            def _(s_idx=s_idx, src=src):
                for kv, gref in enumerate((kg_ref, vg_ref)):
                    pltpu.make_async_remote_copy(
                        src_ref=gref.at[s_idx],
                        dst_ref=gref.at[s_idx],
                        send_sem=kv_send_sems.at[kv, 0],
                        recv_sem=kv_recv_sems.at[kv, s_idx],
                        device_id=(src,),
                        device_id_type=pl.DeviceIdType.MESH,
                    ).wait_recv()

        iq = lax.broadcasted_iota(jnp.int32, (SQ, SKV_SH), 0) // 64
        ik = lax.broadcasted_iota(jnp.int32, (SQ, SKV_SH), 1) // 64
        msk = iq == ik

        ctx_rows = []
        for b in range(B):
            ctx_cols = []
            for h in range(H_SH):
                q = q2[b * SQ:(b + 1) * SQ, h * DH:(h + 1) * DH]
                srows = slice(b * SKV_SH, (b + 1) * SKV_SH)
                scols = slice(h * DH, (h + 1) * DH)
                k0 = kg_ref[0, srows, scols]
                k1 = kg_ref[1, srows, scols]
                s0 = lax.dot_general(q, k0, (((1,), (1,)), ((), ())),
                                     preferred_element_type=jnp.float32) * 0.125
                s1 = lax.dot_general(q, k1, (((1,), (1,)), ((), ())),
                                     preferred_element_type=jnp.float32) * 0.125
                s0 = jnp.where(msk, s0, NEG)
                s1 = jnp.where(msk, s1, NEG)
                m = jnp.maximum(jnp.max(s0, axis=1, keepdims=True),
                                jnp.max(s1, axis=1, keepdims=True))
                e0 = jnp.exp(s0 - m)
                e1 = jnp.exp(s1 - m)
                l = (jnp.sum(e0, axis=1, keepdims=True)
                     + jnp.sum(e1, axis=1, keepdims=True))
                w0 = e0 / l
                w1 = e1 / l
                v0 = vg_ref[0, srows, scols]
                v1 = vg_ref[1, srows, scols]
                ctx = (jnp.dot(w0, v0, preferred_element_type=jnp.float32)
                       + jnp.dot(w1, v1, preferred_element_type=jnp.float32))
                ctx_cols.append(ctx)
            ctx_rows.append(jnp.concatenate(ctx_cols, axis=1))
        ctx2 = jnp.concatenate(ctx_rows, axis=0)

        out_ref[:] = jnp.dot(ctx2, wo_ref[:], preferred_element_type=jnp.float32)

        for s_idx, src in enumerate(SRC_DEVS):
            @pl.when(my == src)
            def _(s_idx=s_idx, src=src):
                d_i = 0
                for dst in range(N_DEV):
                    if dst == src:
                        continue
                    for kv, (ref, gref) in enumerate(
                            ((k_ref, kg_ref), (v_ref, vg_ref))):
                        pltpu.make_async_remote_copy(
                            src_ref=ref.at[:, pl.ds(dst * HCOLS, HCOLS)],
                            dst_ref=gref.at[s_idx],
                            send_sem=kv_send_sems.at[kv, d_i],
                            recv_sem=kv_recv_sems.at[kv, s_idx],
                            device_id=(dst,),
                            device_id_type=pl.DeviceIdType.MESH,
                        ).wait_send()
                    d_i += 1

        for off in range(1, N_DEV):
            pltpu.make_async_remote_copy(
                src_ref=out_ref,
                dst_ref=pg_ref.at[N_DEV - 1 - off],
                send_sem=p_send_sems.at[off - 1],
                recv_sem=p_recv_sems.at[N_DEV - 1 - off],
                device_id=(lax.rem(my + off, N_DEV),),
                device_id_type=pl.DeviceIdType.MESH,
            ).start()
        for slot in range(N_DEV - 1):
            pltpu.make_async_remote_copy(
                src_ref=out_ref,
                dst_ref=pg_ref.at[slot],
                send_sem=p_send_sems.at[0],
                recv_sem=p_recv_sems.at[slot],
                device_id=(lax.rem(my + 1, N_DEV),),
                device_id_type=pl.DeviceIdType.MESH,
            ).wait_recv()
        for off in range(1, N_DEV):
            pltpu.make_async_remote_copy(
                src_ref=out_ref,
                dst_ref=pg_ref.at[N_DEV - 1 - off],
                send_sem=p_send_sems.at[off - 1],
                recv_sem=p_recv_sems.at[N_DEV - 1 - off],
                device_id=(lax.rem(my + off, N_DEV),),
                device_id_type=pl.DeviceIdType.MESH,
            ).wait_send()

        out_ref[:] = (out_ref[:] + pg_ref[0, :, :]
                      + pg_ref[1, :, :] + pg_ref[2, :, :])

    out2 = pl.pallas_call(
        body,
        out_shape=jax.ShapeDtypeStruct((ROWS, D_MODEL), jnp.float32),
        in_specs=[pl.BlockSpec(memory_space=pltpu.VMEM)] * 5,
        out_specs=pl.BlockSpec(memory_space=pltpu.VMEM),
        scratch_shapes=[
            pltpu.VMEM((2, ROWS, HCOLS), jnp.float32),
            pltpu.VMEM((2, ROWS, HCOLS), jnp.float32),
            pltpu.VMEM((3, ROWS, D_MODEL), jnp.float32),
            pltpu.SemaphoreType.DMA((2, 3)),
            pltpu.SemaphoreType.DMA((2, 2)),
            pltpu.SemaphoreType.DMA((3,)),
            pltpu.SemaphoreType.DMA((3,)),
        ],
        compiler_params=pltpu.CompilerParams(collective_id=0),
    )(x2, Wq, k2, v2, Wo)
    return out2.reshape(B, SQ, D_MODEL)


# device time: 35010 ns/iter; 1.1016x vs baseline; 1.1016x over previous
import jax
import jax.numpy as jnp
from jax import lax
from jax.experimental import pallas as pl
from jax.experimental.pallas import tpu as pltpu

N_DEV = 4
B = 2
SQ = 128
SKV_SH = 128
HQ = 16
H_SH = 4
DH = 64
D_MODEL = 512
ROWS = B * SQ
HCOLS = H_SH * DH
SRC_DEVS = (0, 2)
NEG = -1e9


def kernel(x, Wq, K_ext, V_ext, Wo):
    x2 = x.reshape(ROWS, D_MODEL)
    k2 = K_ext.reshape(B * SKV_SH, HQ * DH)
    v2 = V_ext.reshape(B * SKV_SH, HQ * DH)

    def body(x_ref, wq_ref, k_ref, v_ref, wo_ref, out_ref,
             kg_ref, vg_ref, rs_ref,
             kv_send_sems, kv_recv_sems,
             rs_send_sems, rs_recv_sems, ag_send_sems, ag_recv_sems):
        my = lax.axis_index("i")

        barrier = pltpu.get_barrier_semaphore()
        for src in SRC_DEVS:
            @pl.when(my != src)
            def _(src=src):
                pl.semaphore_signal(barrier, inc=1, device_id=(src,),
                                    device_id_type=pl.DeviceIdType.MESH)
        for src in SRC_DEVS:
            @pl.when(my == src)
            def _():
                pl.semaphore_wait(barrier, N_DEV - 1)

        for s_idx, src in enumerate(SRC_DEVS):
            @pl.when(my == src)
            def _(s_idx=s_idx, src=src):
                kg_ref[s_idx, :, :] = k_ref[:, src * HCOLS:(src + 1) * HCOLS]
                vg_ref[s_idx, :, :] = v_ref[:, src * HCOLS:(src + 1) * HCOLS]
                d_i = 0
                for dst in range(N_DEV):
                    if dst == src:
                        continue
                    for kv, (ref, gref) in enumerate(
                            ((k_ref, kg_ref), (v_ref, vg_ref))):
                        pltpu.make_async_remote_copy(
                            src_ref=ref.at[:, pl.ds(dst * HCOLS, HCOLS)],
                            dst_ref=gref.at[s_idx],
                            send_sem=kv_send_sems.at[kv, d_i],
                            recv_sem=kv_recv_sems.at[kv, s_idx],
                            device_id=(dst,),
                            device_id_type=pl.DeviceIdType.MESH,
                        ).start()
                    d_i += 1

        q2 = jnp.dot(x_ref[:], wq_ref[:], preferred_element_type=jnp.float32)

        for s_idx, src in enumerate(SRC_DEVS):
            @pl.when(my != src)
            def _(s_idx=s_idx, src=src):
                for kv, gref in enumerate((kg_ref, vg_ref)):
                    pltpu.make_async_remote_copy(
                        src_ref=gref.at[s_idx],
                        dst_ref=gref.at[s_idx],
                        send_sem=kv_send_sems.at[kv, 0],
                        recv_sem=kv_recv_sems.at[kv, s_idx],
                        device_id=(src,),
                        device_id_type=pl.DeviceIdType.MESH,
                    ).wait_recv()

        iq = lax.broadcasted_iota(jnp.int32, (SQ, SKV_SH), 0) // 64
        ik = lax.broadcasted_iota(jnp.int32, (SQ, SKV_SH), 1) // 64
        msk = iq == ik

        ctx_rows = []
        for b in range(B):
            ctx_cols = []
            for h in range(H_SH):
                q = q2[b * SQ:(b + 1) * SQ, h * DH:(h + 1) * DH]
                srows = slice(b * SKV_SH, (b + 1) * SKV_SH)
                scols = slice(h * DH, (h + 1) * DH)
                k0 = kg_ref[0, srows, scols]
                k1 = kg_ref[1, srows, scols]
                s0 = lax.dot_general(q, k0, (((1,), (1,)), ((), ())),
                                     preferred_element_type=jnp.float32) * 0.125
                s1 = lax.dot_general(q, k1, (((1,), (1,)), ((), ())),
                                     preferred_element_type=jnp.float32) * 0.125
                s0 = jnp.where(msk, s0, NEG)
                s1 = jnp.where(msk, s1, NEG)
                m = jnp.maximum(jnp.max(s0, axis=1, keepdims=True),
                                jnp.max(s1, axis=1, keepdims=True))
                e0 = jnp.exp(s0 - m)
                e1 = jnp.exp(s1 - m)
                l = (jnp.sum(e0, axis=1, keepdims=True)
                     + jnp.sum(e1, axis=1, keepdims=True))
                w0 = e0 / l
                w1 = e1 / l
                v0 = vg_ref[0, srows, scols]
                v1 = vg_ref[1, srows, scols]
                ctx = (jnp.dot(w0, v0, preferred_element_type=jnp.float32)
                       + jnp.dot(w1, v1, preferred_element_type=jnp.float32))
                ctx_cols.append(ctx)
            ctx_rows.append(jnp.concatenate(ctx_cols, axis=1))
        ctx2 = jnp.concatenate(ctx_rows, axis=0)

        out_ref[:] = jnp.dot(ctx2, wo_ref[:], preferred_element_type=jnp.float32)

        R = ROWS // N_DEV
        for off in range(1, N_DEV):
            dst = lax.rem(my + off, N_DEV)
            pltpu.make_async_remote_copy(
                src_ref=out_ref.at[pl.ds(dst * R, R), :],
                dst_ref=rs_ref.at[off - 1],
                send_sem=rs_send_sems.at[off - 1],
                recv_sem=rs_recv_sems.at[off - 1],
                device_id=(dst,),
                device_id_type=pl.DeviceIdType.MESH,
            ).start()
        for slot in range(N_DEV - 1):
            pltpu.make_async_remote_copy(
                src_ref=rs_ref.at[slot],
                dst_ref=rs_ref.at[slot],
                send_sem=rs_send_sems.at[0],
                recv_sem=rs_recv_sems.at[slot],
                device_id=(lax.rem(my + 1, N_DEV),),
                device_id_type=pl.DeviceIdType.MESH,
            ).wait_recv()

        myrows = pl.ds(my * R, R)
        out_ref[myrows, :] = (out_ref[myrows, :] + rs_ref[0, :, :]
                              + rs_ref[1, :, :] + rs_ref[2, :, :])

        for off in range(1, N_DEV):
            dst = lax.rem(my + off, N_DEV)
            pltpu.make_async_remote_copy(
                src_ref=out_ref.at[myrows, :],
                dst_ref=out_ref.at[myrows, :],
                send_sem=ag_send_sems.at[off - 1],
                recv_sem=ag_recv_sems.at[off - 1],
                device_id=(dst,),
                device_id_type=pl.DeviceIdType.MESH,
            ).start()
        for slot in range(N_DEV - 1):
            j = lax.rem(my + N_DEV - 1 - slot, N_DEV)
            pltpu.make_async_remote_copy(
                src_ref=out_ref.at[pl.ds(j * R, R), :],
                dst_ref=out_ref.at[pl.ds(j * R, R), :],
                send_sem=ag_send_sems.at[0],
                recv_sem=ag_recv_sems.at[slot],
                device_id=(j,),
                device_id_type=pl.DeviceIdType.MESH,
            ).wait_recv()

        for off in range(1, N_DEV):
            dst = lax.rem(my + off, N_DEV)
            pltpu.make_async_remote_copy(
                src_ref=out_ref.at[pl.ds(dst * R, R), :],
                dst_ref=rs_ref.at[off - 1],
                send_sem=rs_send_sems.at[off - 1],
                recv_sem=rs_recv_sems.at[off - 1],
                device_id=(dst,),
                device_id_type=pl.DeviceIdType.MESH,
            ).wait_send()
            pltpu.make_async_remote_copy(
                src_ref=out_ref.at[myrows, :],
                dst_ref=out_ref.at[myrows, :],
                send_sem=ag_send_sems.at[off - 1],
                recv_sem=ag_recv_sems.at[off - 1],
                device_id=(dst,),
                device_id_type=pl.DeviceIdType.MESH,
            ).wait_send()
        for s_idx, src in enumerate(SRC_DEVS):
            @pl.when(my == src)
            def _(s_idx=s_idx, src=src):
                d_i = 0
                for dst in range(N_DEV):
                    if dst == src:
                        continue
                    for kv, (ref, gref) in enumerate(
                            ((k_ref, kg_ref), (v_ref, vg_ref))):
                        pltpu.make_async_remote_copy(
                            src_ref=ref.at[:, pl.ds(dst * HCOLS, HCOLS)],
                            dst_ref=gref.at[s_idx],
                            send_sem=kv_send_sems.at[kv, d_i],
                            recv_sem=kv_recv_sems.at[kv, s_idx],
                            device_id=(dst,),
                            device_id_type=pl.DeviceIdType.MESH,
                        ).wait_send()
                    d_i += 1

    out2 = pl.pallas_call(
        body,
        out_shape=jax.ShapeDtypeStruct((ROWS, D_MODEL), jnp.float32),
        in_specs=[pl.BlockSpec(memory_space=pltpu.VMEM)] * 5,
        out_specs=pl.BlockSpec(memory_space=pltpu.VMEM),
        scratch_shapes=[
            pltpu.VMEM((2, ROWS, HCOLS), jnp.float32),
            pltpu.VMEM((2, ROWS, HCOLS), jnp.float32),
            pltpu.VMEM((3, ROWS // N_DEV, D_MODEL), jnp.float32),
            pltpu.SemaphoreType.DMA((2, 3)),
            pltpu.SemaphoreType.DMA((2, 2)),
            pltpu.SemaphoreType.DMA((3,)),
            pltpu.SemaphoreType.DMA((3,)),
            pltpu.SemaphoreType.DMA((3,)),
            pltpu.SemaphoreType.DMA((3,)),
        ],
        compiler_params=pltpu.CompilerParams(collective_id=0),
    )(x2, Wq, k2, v2, Wo)
    return out2.reshape(B, SQ, D_MODEL)


# device time: 31647 ns/iter; 1.2186x vs baseline; 1.1063x over previous
import contextlib
import os

import jax
import jax.numpy as jnp
from jax import lax

_SCOPES = os.environ.get("KERNEL_SCOPES", "0") == "1"
_PROBE = os.environ.get("KERNEL_PROBE", "")


def _scope(name):
    return jax.named_scope(name) if _SCOPES else contextlib.nullcontext()
from jax.experimental import pallas as pl
from jax.experimental.pallas import tpu as pltpu

N_DEV = 4
B = 2
SQ = 128
SKV_SH = 128
HQ = 16
H_SH = 4
DH = 64
D_MODEL = 512
ROWS = B * SQ
HCOLS = H_SH * DH
SRC_DEVS = (0, 2)
NEG = -1e9


def kernel(x, Wq, K_ext, V_ext, Wo):
    x2 = x.reshape(ROWS, D_MODEL)
    k2 = K_ext.reshape(B * SKV_SH, HQ * DH)
    v2 = V_ext.reshape(B * SKV_SH, HQ * DH)

    def body(x_ref, wq_ref, k_ref, v_ref, wo_ref, out_ref,
             kg_ref, vg_ref, rs_ref,
             kv_send_sems, kv_recv_sems,
             rs_send_sems, rs_recv_sems, ag_send_sems, ag_recv_sems):
        my = lax.axis_index("i")

        with _scope("phase_barrier"):
            barrier = pltpu.get_barrier_semaphore()
            for src in SRC_DEVS:
                @pl.when(my != src)
                def _(src=src):
                    pl.semaphore_signal(barrier, inc=1, device_id=(src,),
                                        device_id_type=pl.DeviceIdType.MESH)
            for src in SRC_DEVS:
                @pl.when(my == src)
                def _():
                    pl.semaphore_wait(barrier, N_DEV - 1)

        with _scope("phase_p1_send"):
            for s_idx, src in enumerate(SRC_DEVS if _PROBE != "nop1" else ()):
                @pl.when(my == src)
                def _(s_idx=s_idx, src=src):
                    kg_ref[s_idx, :, :] = k_ref[:, src * HCOLS:(src + 1) * HCOLS]
                    vg_ref[s_idx, :, :] = v_ref[:, src * HCOLS:(src + 1) * HCOLS]
                    d_i = 0
                    for dst in range(N_DEV):
                        if dst == src:
                            continue
                        for kv, (ref, gref) in enumerate(
                                ((k_ref, kg_ref), (v_ref, vg_ref))):
                            pltpu.make_async_remote_copy(
                                src_ref=ref.at[:, pl.ds(dst * HCOLS, HCOLS)],
                                dst_ref=gref.at[s_idx],
                                send_sem=kv_send_sems.at[kv, d_i],
                                recv_sem=kv_recv_sems.at[kv, s_idx],
                                device_id=(dst,),
                                device_id_type=pl.DeviceIdType.MESH,
                            ).start()
                        d_i += 1

        with _scope("phase_qproj"):
            q2 = jnp.dot(x_ref[:], wq_ref[:],
                         preferred_element_type=jnp.float32)

        with _scope("phase_p1_wait"):
            for s_idx, src in enumerate(SRC_DEVS if _PROBE != "nop1" else ()):
                @pl.when(my != src)
                def _(s_idx=s_idx, src=src):
                    for kv, gref in enumerate((kg_ref, vg_ref)):
                        pltpu.make_async_remote_copy(
                            src_ref=gref.at[s_idx],
                            dst_ref=gref.at[s_idx],
                            send_sem=kv_send_sems.at[kv, 0],
                            recv_sem=kv_recv_sems.at[kv, s_idx],
                            device_id=(src,),
                            device_id_type=pl.DeviceIdType.MESH,
                        ).wait_recv()

        with _scope("phase_attn"):
            iq = lax.broadcasted_iota(jnp.int32, (SQ, SKV_SH), 0) // 64
            ik = lax.broadcasted_iota(jnp.int32, (SQ, SKV_SH), 1) // 64
            msk = iq == ik

            ctx_rows = []
            for b in range(B if _PROBE != "noattn" else 0):
                ctx_cols = []
                for h in range(H_SH):
                    q = q2[b * SQ:(b + 1) * SQ, h * DH:(h + 1) * DH]
                    srows = slice(b * SKV_SH, (b + 1) * SKV_SH)
                    scols = slice(h * DH, (h + 1) * DH)
                    k0 = kg_ref[0, srows, scols]
                    k1 = kg_ref[1, srows, scols]
                    s0 = lax.dot_general(q, k0, (((1,), (1,)), ((), ())),
                                         preferred_element_type=jnp.float32) * 0.125
                    s1 = lax.dot_general(q, k1, (((1,), (1,)), ((), ())),
                                         preferred_element_type=jnp.float32) * 0.125
                    s0 = jnp.where(msk, s0, NEG)
                    s1 = jnp.where(msk, s1, NEG)
                    m = jnp.maximum(jnp.max(s0, axis=1, keepdims=True),
                                    jnp.max(s1, axis=1, keepdims=True))
                    e0 = jnp.exp(s0 - m)
                    e1 = jnp.exp(s1 - m)
                    l = (jnp.sum(e0, axis=1, keepdims=True)
                         + jnp.sum(e1, axis=1, keepdims=True))
                    w0 = e0 / l
                    w1 = e1 / l
                    v0 = vg_ref[0, srows, scols]
                    v1 = vg_ref[1, srows, scols]
                    ctx = (jnp.dot(w0, v0, preferred_element_type=jnp.float32)
                           + jnp.dot(w1, v1, preferred_element_type=jnp.float32))
                    ctx_cols.append(ctx)
                ctx_rows.append(jnp.concatenate(ctx_cols, axis=1))
            ctx2 = (jnp.concatenate(ctx_rows, axis=0)
                    if _PROBE != "noattn" else q2)

        with _scope("phase_gemm"):
            out_ref[:] = jnp.dot(ctx2, wo_ref[:],
                                 preferred_element_type=jnp.float32)

        R = ROWS // N_DEV
        P2_OFFS = range(1, N_DEV) if _PROBE != "nop2" else range(0)
        P2_SLOTS = range(N_DEV - 1) if _PROBE != "nop2" else range(0)
        with _scope("phase_rs_send"):
            for off in P2_OFFS:
                dst = lax.rem(my + off, N_DEV)
                pltpu.make_async_remote_copy(
                    src_ref=out_ref.at[pl.ds(dst * R, R), :],
                    dst_ref=rs_ref.at[off - 1],
                    send_sem=rs_send_sems.at[off - 1],
                    recv_sem=rs_recv_sems.at[off - 1],
                    device_id=(dst,),
                    device_id_type=pl.DeviceIdType.MESH,
                ).start()
        with _scope("phase_rs_wait"):
            for slot in P2_SLOTS:
                pltpu.make_async_remote_copy(
                    src_ref=rs_ref.at[slot],
                    dst_ref=rs_ref.at[slot],
                    send_sem=rs_send_sems.at[0],
                    recv_sem=rs_recv_sems.at[slot],
                    device_id=(lax.rem(my + 1, N_DEV),),
                    device_id_type=pl.DeviceIdType.MESH,
                ).wait_recv()

        with _scope("phase_reduce"):
            myrows = pl.ds(my * R, R)
            if _PROBE != "nop2":
                out_ref[myrows, :] = (out_ref[myrows, :] + rs_ref[0, :, :]
                                      + rs_ref[1, :, :] + rs_ref[2, :, :])

        with _scope("phase_ag_send"):
            for off in P2_OFFS:
                dst = lax.rem(my + off, N_DEV)
                pltpu.make_async_remote_copy(
                    src_ref=out_ref.at[myrows, :],
                    dst_ref=out_ref.at[myrows, :],
                    send_sem=ag_send_sems.at[off - 1],
                    recv_sem=ag_recv_sems.at[off - 1],
                    device_id=(dst,),
                    device_id_type=pl.DeviceIdType.MESH,
                ).start()
        with _scope("phase_ag_wait"):
            for slot in P2_SLOTS:
                j = lax.rem(my + N_DEV - 1 - slot, N_DEV)
                pltpu.make_async_remote_copy(
                    src_ref=out_ref.at[pl.ds(j * R, R), :],
                    dst_ref=out_ref.at[pl.ds(j * R, R), :],
                    send_sem=ag_send_sems.at[0],
                    recv_sem=ag_recv_sems.at[slot],
                    device_id=(j,),
                    device_id_type=pl.DeviceIdType.MESH,
                ).wait_recv()

        for off in P2_OFFS:
            dst = lax.rem(my + off, N_DEV)
            pltpu.make_async_remote_copy(
                src_ref=out_ref.at[pl.ds(dst * R, R), :],
                dst_ref=rs_ref.at[off - 1],
                send_sem=rs_send_sems.at[off - 1],
                recv_sem=rs_recv_sems.at[off - 1],
                device_id=(dst,),
                device_id_type=pl.DeviceIdType.MESH,
            ).wait_send()
            pltpu.make_async_remote_copy(
                src_ref=out_ref.at[myrows, :],
                dst_ref=out_ref.at[myrows, :],
                send_sem=ag_send_sems.at[off - 1],
                recv_sem=ag_recv_sems.at[off - 1],
                device_id=(dst,),
                device_id_type=pl.DeviceIdType.MESH,
            ).wait_send()
        for s_idx, src in enumerate(SRC_DEVS if _PROBE != "nop1" else ()):
            @pl.when(my == src)
            def _(s_idx=s_idx, src=src):
                d_i = 0
                for dst in range(N_DEV):
                    if dst == src:
                        continue
                    for kv, (ref, gref) in enumerate(
                            ((k_ref, kg_ref), (v_ref, vg_ref))):
                        pltpu.make_async_remote_copy(
                            src_ref=ref.at[:, pl.ds(dst * HCOLS, HCOLS)],
                            dst_ref=gref.at[s_idx],
                            send_sem=kv_send_sems.at[kv, d_i],
                            recv_sem=kv_recv_sems.at[kv, s_idx],
                            device_id=(dst,),
                            device_id_type=pl.DeviceIdType.MESH,
                        ).wait_send()
                    d_i += 1

    out2 = pl.pallas_call(
        body,
        out_shape=jax.ShapeDtypeStruct((ROWS, D_MODEL), jnp.float32),
        in_specs=[pl.BlockSpec(memory_space=pltpu.VMEM)] * 5,
        out_specs=pl.BlockSpec(memory_space=pltpu.VMEM),
        scratch_shapes=[
            pltpu.VMEM((2, ROWS, HCOLS), jnp.float32),
            pltpu.VMEM((2, ROWS, HCOLS), jnp.float32),
            pltpu.VMEM((3, ROWS // N_DEV, D_MODEL), jnp.float32),
            pltpu.SemaphoreType.DMA((2, 3)),
            pltpu.SemaphoreType.DMA((2, 2)),
            pltpu.SemaphoreType.DMA((3,)),
            pltpu.SemaphoreType.DMA((3,)),
            pltpu.SemaphoreType.DMA((3,)),
            pltpu.SemaphoreType.DMA((3,)),
        ],
        compiler_params=pltpu.CompilerParams(collective_id=0),
    )(x2, Wq, k2, v2, Wo)
    return out2.reshape(B, SQ, D_MODEL)
